# baseline (speedup 1.0000x reference)
import functools

import jax
import jax.numpy as jnp
from jax import lax
from jax.experimental import pallas as pl
from jax.experimental.pallas import tpu as pltpu
from jax.experimental.pallas import tpu_sc as plsc

_NC, _NS = 2, 16
_NW = _NC * _NS


def kernel(x, pe):
    seq_len = x.shape[1]
    d = pe.shape[1]
    rows_w = seq_len // _NW

    mesh = plsc.VectorSubcoreMesh(core_axis_name="c", subcore_axis_name="s")

    @functools.partial(
        pl.kernel,
        mesh=mesh,
        out_type=jax.ShapeDtypeStruct((seq_len, d), jnp.float32),
    )
    def copy_k(pe_hbm, out_hbm):
        wid = lax.axis_index("s") * _NC + lax.axis_index("c")
        base = wid * rows_w
        pltpu.sync_copy(pe_hbm.at[pl.ds(base, rows_w)],
                        out_hbm.at[pl.ds(base, rows_w)])

    return copy_k(pe)[None]


# speedup vs baseline: 24.7821x; 24.7821x over previous
import functools

import jax
import jax.numpy as jnp
from jax import lax
from jax.experimental import pallas as pl
from jax.experimental.pallas import tpu as pltpu
from jax.experimental.pallas import tpu_sc as plsc

_NC, _NS = 2, 16
_NW = _NC * _NS
_CHUNK = 32
_NB = 3


def kernel(x, pe):
    seq_len = x.shape[1]
    d = pe.shape[1]
    rows_w = seq_len // _NW
    nchunks = rows_w // _CHUNK

    mesh = plsc.VectorSubcoreMesh(core_axis_name="c", subcore_axis_name="s")

    @functools.partial(
        pl.kernel,
        mesh=mesh,
        out_type=jax.ShapeDtypeStruct((seq_len, d), jnp.float32),
        scratch_types=[
            pltpu.VMEM((_NB, _CHUNK, d), jnp.float32),
            pltpu.SemaphoreType.DMA,
            pltpu.SemaphoreType.DMA,
            pltpu.SemaphoreType.DMA,
            pltpu.SemaphoreType.DMA,
            pltpu.SemaphoreType.DMA,
            pltpu.SemaphoreType.DMA,
        ],
    )
    def copy_k(pe_hbm, out_hbm, buf, si0, si1, si2, so0, so1, so2):
        wid = lax.axis_index("s") * _NC + lax.axis_index("c")
        base = wid * rows_w
        s_in = (si0, si1, si2)
        s_out = (so0, so1, so2)

        def in_copy(i):
            b = i % _NB
            return pltpu.make_async_copy(
                pe_hbm.at[pl.ds(base + i * _CHUNK, _CHUNK)], buf.at[b], s_in[b])

        def out_copy(i):
            b = i % _NB
            return pltpu.make_async_copy(
                buf.at[b], out_hbm.at[pl.ds(base + i * _CHUNK, _CHUNK)], s_out[b])

        for j in range(min(_NB, nchunks)):
            in_copy(j).start()
        for i in range(nchunks):
            in_copy(i).wait()
            out_copy(i).start()
            nxt = i + _NB
            if nxt < nchunks:
                out_copy(i).wait()
                in_copy(nxt).start()
        for i in range(max(nchunks - _NB, 0), nchunks):
            out_copy(i).wait()

    return copy_k(pe)[None]
